# baseline (speedup 1.0000x reference)
import functools

import jax
import jax.numpy as jnp
from jax import lax
from jax.experimental import pallas as pl
from jax.experimental.pallas import tpu as pltpu
from jax.experimental.pallas import tpu_sc as plsc

N_USERS = 100000
N_ITEMS = 100000
D = 64
B = 4096
L = 50
LP = 56
NW = 32
ROWS_W = B // NW
CHUNK = 2
NCHUNK = ROWS_W // CHUNK


def _proj_body(u_ref, w1_ref, r_ref, w2_ref, bh_ref, pu_ref, prb_ref):
    pu_ref[...] = jnp.dot(u_ref[...], w1_ref[...],
                          preferred_element_type=jnp.float32)
    prb_ref[...] = jnp.dot(r_ref[...], w2_ref[...],
                           preferred_element_type=jnp.float32) + bh_ref[...]


def _project_tables(u2e, wh1, r2e_pad, wh2, bh_row):
    blk = 4000
    grid = N_USERS // blk
    return pl.pallas_call(
        _proj_body,
        grid=(grid,),
        in_specs=[
            pl.BlockSpec((blk, D), lambda i: (i, 0)),
            pl.BlockSpec((D, D), lambda i: (0, 0)),
            pl.BlockSpec((8, D), lambda i: (0, 0)),
            pl.BlockSpec((D, D), lambda i: (0, 0)),
            pl.BlockSpec((1, D), lambda i: (0, 0)),
        ],
        out_specs=[
            pl.BlockSpec((blk, D), lambda i: (i, 0)),
            pl.BlockSpec((8, D), lambda i: (0, 0)),
        ],
        out_shape=[
            jax.ShapeDtypeStruct((N_USERS, D), jnp.float32),
            jax.ShapeDtypeStruct((8, D), jnp.float32),
        ],
    )(u2e, wh1, r2e_pad, wh2, bh_row)


def _sc_body(pu_hbm, v2e_hbm, prb_hbm, idxh_hbm, idxr_hbm, idxa_hbm, nodes_hbm,
             hist_out_hbm, adj_out_hbm, self_out_hbm,
             idxh_v, idxr_v, idxa_v, idxn_v,
             hbuf0, hbuf1, abuf0, abuf1, pbuf0, pbuf1, sbuf, hist_o, adj_o,
             semh0, semh1, sema0, sema1, semp0, semp1, sems):
    wid = lax.axis_index("s") * 2 + lax.axis_index("c")
    row0 = wid * ROWS_W

    pltpu.sync_copy(idxh_hbm.at[pl.ds(wid * (ROWS_W // CHUNK), NCHUNK)], idxh_v)
    pltpu.sync_copy(idxa_hbm.at[pl.ds(wid * (ROWS_W // CHUNK), NCHUNK)], idxa_v)
    pltpu.sync_copy(idxr_hbm.at[pl.ds(wid * (ROWS_W // CHUNK), NCHUNK)], idxr_v)
    pltpu.sync_copy(nodes_hbm.at[pl.ds(row0, ROWS_W)], idxn_v)

    self_cp = pltpu.async_copy(v2e_hbm.at[idxn_v], sbuf, sems)

    inv_l = jnp.float32(1.0 / L)
    zero = jnp.zeros((16,), jnp.float32)

    def issue(c, hb, ab, pb, sh, sa, sp):
        pltpu.async_copy(pu_hbm.at[idxh_v.at[c]], hb, sh)
        pltpu.async_copy(v2e_hbm.at[idxa_v.at[c]], ab, sa)
        pltpu.async_copy(prb_hbm.at[idxr_v.at[c]], pb, sp)

    def wait(c, hb, ab, pb, sh, sa, sp):
        pltpu.make_async_copy(pu_hbm.at[idxh_v.at[c]], hb, sh).wait()
        pltpu.make_async_copy(v2e_hbm.at[idxa_v.at[c]], ab, sa).wait()
        pltpu.make_async_copy(prb_hbm.at[idxr_v.at[c]], pb, sp).wait()

    def compute_chunk(c, hbuf, abuf, pbuf):
        for r2 in range(CHUNK):
            row_l = c * CHUNK + r2

            @plsc.parallel_loop(r2 * LP, r2 * LP + L, carry=(zero,) * 8,
                                unroll=2)
            def accs(o, acc):
                new = []
                for j in range(4):
                    g = hbuf[o, pl.ds(16 * j, 16)]
                    p = pbuf[o, pl.ds(16 * j, 16)]
                    new.append(acc[j] + jnp.maximum(g + p, 0.0))
                for j in range(4):
                    new.append(acc[4 + j] + abuf[o, pl.ds(16 * j, 16)])
                return tuple(new)

            for j in range(4):
                hist_o[row_l, pl.ds(16 * j, 16)] = accs[j] * inv_l
                adj_o[row_l, pl.ds(16 * j, 16)] = accs[4 + j] * inv_l

    issue(0, hbuf0, abuf0, pbuf0, semh0, sema0, semp0)

    def pair_loop(i, carry):
        c0 = 2 * i
        c1 = 2 * i + 1
        issue(c1, hbuf1, abuf1, pbuf1, semh1, sema1, semp1)
        wait(c0, hbuf0, abuf0, pbuf0, semh0, sema0, semp0)
        compute_chunk(c0, hbuf0, abuf0, pbuf0)

        @pl.when(c1 + 1 < NCHUNK)
        def _():
            issue(c1 + 1, hbuf0, abuf0, pbuf0, semh0, sema0, semp0)

        wait(c1, hbuf1, abuf1, pbuf1, semh1, sema1, semp1)
        compute_chunk(c1, hbuf1, abuf1, pbuf1)
        return carry

    lax.fori_loop(0, NCHUNK // 2, pair_loop, 0)

    self_cp.wait()
    pltpu.sync_copy(sbuf, self_out_hbm.at[pl.ds(row0, ROWS_W)])
    pltpu.sync_copy(hist_o, hist_out_hbm.at[pl.ds(row0, ROWS_W)])
    pltpu.sync_copy(adj_o, adj_out_hbm.at[pl.ds(row0, ROWS_W)])


def _sc_gather_agg(pu, v2e, prb, idxh, idxr, idxa, nodes):
    mesh = plsc.VectorSubcoreMesh(core_axis_name="c", subcore_axis_name="s")
    f32 = jnp.float32
    kern = functools.partial(
        pl.kernel,
        mesh=mesh,
        compiler_params=pltpu.CompilerParams(use_tc_tiling_on_sc=False),
        out_type=[
            jax.ShapeDtypeStruct((B, D), f32),
            jax.ShapeDtypeStruct((B, D), f32),
            jax.ShapeDtypeStruct((B, D), f32),
        ],
        scratch_types=[
            pltpu.VMEM((NCHUNK, CHUNK * LP), jnp.int32),
            pltpu.VMEM((NCHUNK, CHUNK * LP), jnp.int32),
            pltpu.VMEM((NCHUNK, CHUNK * LP), jnp.int32),
            pltpu.VMEM((ROWS_W,), jnp.int32),
            pltpu.VMEM((CHUNK * LP, D), f32),
            pltpu.VMEM((CHUNK * LP, D), f32),
            pltpu.VMEM((CHUNK * LP, D), f32),
            pltpu.VMEM((CHUNK * LP, D), f32),
            pltpu.VMEM((CHUNK * LP, D), f32),
            pltpu.VMEM((CHUNK * LP, D), f32),
            pltpu.VMEM((ROWS_W, D), f32),
            pltpu.VMEM((ROWS_W, D), f32),
            pltpu.VMEM((ROWS_W, D), f32),
            pltpu.SemaphoreType.DMA,
            pltpu.SemaphoreType.DMA,
            pltpu.SemaphoreType.DMA,
            pltpu.SemaphoreType.DMA,
            pltpu.SemaphoreType.DMA,
            pltpu.SemaphoreType.DMA,
            pltpu.SemaphoreType.DMA,
        ],
    )(_sc_body)
    return kern(pu, v2e, prb, idxh, idxr, idxa, nodes)


def _combine_body(self_ref, hist_ref, adj_ref, ws_ref, bs_ref,
                  w1a_ref, w1b_ref, b1_ref, out_ref):
    soc = jnp.maximum(
        jnp.dot(adj_ref[...], ws_ref[...], preferred_element_type=jnp.float32)
        + bs_ref[...], 0.0)
    neigh = 0.5 * (hist_ref[...] + soc)
    out = (jnp.dot(self_ref[...], w1a_ref[...], preferred_element_type=jnp.float32)
           + jnp.dot(neigh, w1b_ref[...], preferred_element_type=jnp.float32)
           + b1_ref[...])
    out_ref[...] = jnp.maximum(out, 0.0)


def _combine(self_feats, hist_agg, adj_mean, Ws, bs_row, w1a, w1b, b1_row):
    return pl.pallas_call(
        _combine_body,
        out_shape=jax.ShapeDtypeStruct((B, D), jnp.float32),
    )(self_feats, hist_agg, adj_mean, Ws, bs_row, w1a, w1b, b1_row)


def kernel(nodes, nodes_target, hist_uv, hist_r, adj, u2e, v2e, r2e,
           Wh, bh, Ws, bs, W1, b1):
    del nodes_target

    wh1 = Wh[:D]
    wh2 = Wh[D:]
    w1a = W1[:D]
    w1b = W1[D:]
    r2e_pad = jnp.concatenate(
        [r2e, jnp.zeros((8 - r2e.shape[0], D), jnp.float32)], axis=0)
    bh_row = bh.reshape(1, D)
    bs_row = bs.reshape(1, D)
    b1_row = b1.reshape(1, D)

    pu, prb = _project_tables(u2e, wh1, r2e_pad, wh2, bh_row)

    def pad_lp(a):
        a = a.astype(jnp.int32)
        return jnp.pad(a, ((0, 0), (0, LP - L)))

    idxh = pad_lp(hist_uv).reshape(B // CHUNK, CHUNK * LP)
    idxa = pad_lp(adj).reshape(B // CHUNK, CHUNK * LP)
    idxr = pad_lp(hist_r).reshape(B // CHUNK, CHUNK * LP)
    nodes_i = nodes.astype(jnp.int32)

    hist_agg, adj_mean, self_feats = _sc_gather_agg(
        pu, v2e, prb, idxh, idxr, idxa, nodes_i)

    return _combine(self_feats, hist_agg, adj_mean, Ws, bs_row, w1a, w1b, b1_row)


# speedup vs baseline: 2.4920x; 2.4920x over previous
import functools

import jax
import jax.numpy as jnp
from jax import lax
from jax.experimental import pallas as pl
from jax.experimental.pallas import tpu as pltpu
from jax.experimental.pallas import tpu_sc as plsc

N_USERS = 100000
N_ITEMS = 100000
D = 64
B = 4096
L = 50
LP = 56
NW = 32
ROWS_W = B // NW
CHUNK = 2
NCHUNK = ROWS_W // CHUNK


def _proj_body(u_ref, w1_ref, r_ref, w2_ref, bh_ref, pu_ref, prb_ref):
    pu_ref[...] = jnp.dot(u_ref[...], w1_ref[...],
                          preferred_element_type=jnp.float32
                          ).astype(jnp.bfloat16)
    prb_ref[...] = (jnp.dot(r_ref[...], w2_ref[...],
                            preferred_element_type=jnp.float32)
                    + bh_ref[...]).astype(jnp.bfloat16)


def _project_tables(u2e, wh1, r2e_pad, wh2, bh_row):
    blk = 4000
    grid = N_USERS // blk
    return pl.pallas_call(
        _proj_body,
        grid=(grid,),
        in_specs=[
            pl.BlockSpec((blk, D), lambda i: (i, 0)),
            pl.BlockSpec((D, D), lambda i: (0, 0)),
            pl.BlockSpec((8, D), lambda i: (0, 0)),
            pl.BlockSpec((D, D), lambda i: (0, 0)),
            pl.BlockSpec((1, D), lambda i: (0, 0)),
        ],
        out_specs=[
            pl.BlockSpec((blk, D), lambda i: (i, 0)),
            pl.BlockSpec((8, D), lambda i: (0, 0)),
        ],
        out_shape=[
            jax.ShapeDtypeStruct((N_USERS, D), jnp.bfloat16),
            jax.ShapeDtypeStruct((8, D), jnp.bfloat16),
        ],
    )(u2e, wh1, r2e_pad, wh2, bh_row)


def _sc_body(pu_hbm, v2e_hbm, prb_hbm, idxh_hbm, idxr_hbm, idxa_hbm, nodes_hbm,
             hist_out_hbm, adj_out_hbm, self_out_hbm,
             idxh_v, idxr_v, idxa_v, idxn_v, prb_v,
             hbuf0, hbuf1, abuf0, abuf1, sbuf, hist_o, adj_o,
             semh0, semh1, sema0, sema1, sems):
    wid = lax.axis_index("s") * 2 + lax.axis_index("c")
    row0 = wid * ROWS_W

    pltpu.sync_copy(idxh_hbm.at[pl.ds(wid * (ROWS_W // CHUNK), NCHUNK)], idxh_v)
    pltpu.sync_copy(idxa_hbm.at[pl.ds(wid * (ROWS_W // CHUNK), NCHUNK)], idxa_v)
    pltpu.sync_copy(idxr_hbm.at[pl.ds(wid * (ROWS_W // CHUNK), NCHUNK)], idxr_v)
    pltpu.sync_copy(nodes_hbm.at[pl.ds(row0, ROWS_W)], idxn_v)
    pltpu.sync_copy(prb_hbm, prb_v)

    self_cp = pltpu.async_copy(v2e_hbm.at[idxn_v], sbuf, sems)

    inv_l = jnp.float32(1.0 / L)
    zero = jnp.zeros((16,), jnp.float32)
    bf16 = jnp.bfloat16
    ilv = plsc.PackFormat.INTERLEAVED

    def unpk(w):
        return plsc.unpack(plsc.bitcast(w, bf16), format=ilv,
                           preferred_element_type=jnp.float32)

    def issue(c, hb, ab, sh, sa):
        pltpu.async_copy(pu_hbm.at[idxh_v.at[c]], hb, sh)
        pltpu.async_copy(v2e_hbm.at[idxa_v.at[c]], ab, sa)

    def wait(c, hb, ab, sh, sa):
        pltpu.make_async_copy(pu_hbm.at[idxh_v.at[c]], hb, sh).wait()
        pltpu.make_async_copy(v2e_hbm.at[idxa_v.at[c]], ab, sa).wait()

    iota16 = lax.iota(jnp.int32, 16)
    cjs = [iota16 + 16 * j for j in range(2)]

    def compute_chunk(c, hbuf, abuf):
        for r2 in range(CHUNK):
            row_l = c * CHUNK + r2

            @plsc.parallel_loop(r2 * LP, r2 * LP + L, carry=(zero,) * 8,
                                unroll=2)
            def accs(o, acc):
                grp = (o // 16) * 16
                lane = o - grp
                rgrp = idxr_v[c, pl.ds(pl.multiple_of(grp, 8), 16)]
                r32 = rgrp.at[jnp.full((16,), lane, jnp.int32)].get(
                    mode="promise_in_bounds")
                new = list(acc)
                for j in range(2):
                    ge, go = unpk(hbuf[o, pl.ds(16 * j, 16)])
                    pe, po = unpk(plsc.load_gather(prb_v, [r32 + cjs[j]]))
                    ae, ao = unpk(abuf[o, pl.ds(16 * j, 16)])
                    new[2 * j] = acc[2 * j] + jnp.maximum(ge + pe, 0.0)
                    new[2 * j + 1] = acc[2 * j + 1] + jnp.maximum(go + po, 0.0)
                    new[4 + 2 * j] = acc[4 + 2 * j] + ae
                    new[5 + 2 * j] = acc[5 + 2 * j] + ao
                return tuple(new)

            for j in range(2):
                hist_o[row_l, pl.ds(16 * j, 16)] = plsc.bitcast(
                    plsc.pack(accs[2 * j] * inv_l, accs[2 * j + 1] * inv_l,
                              format=ilv), jnp.int32)
                adj_o[row_l, pl.ds(16 * j, 16)] = plsc.bitcast(
                    plsc.pack(accs[4 + 2 * j] * inv_l, accs[5 + 2 * j] * inv_l,
                              format=ilv), jnp.int32)

    issue(0, hbuf0, abuf0, semh0, sema0)

    def pair_loop(i, carry):
        c0 = 2 * i
        c1 = 2 * i + 1
        issue(c1, hbuf1, abuf1, semh1, sema1)
        wait(c0, hbuf0, abuf0, semh0, sema0)
        compute_chunk(c0, hbuf0, abuf0)

        @pl.when(c1 + 1 < NCHUNK)
        def _():
            issue(c1 + 1, hbuf0, abuf0, semh0, sema0)

        wait(c1, hbuf1, abuf1, semh1, sema1)
        compute_chunk(c1, hbuf1, abuf1)
        return carry

    lax.fori_loop(0, NCHUNK // 2, pair_loop, 0)

    self_cp.wait()
    pltpu.sync_copy(sbuf, self_out_hbm.at[pl.ds(row0, ROWS_W)])
    pltpu.sync_copy(hist_o, hist_out_hbm.at[pl.ds(row0, ROWS_W)])
    pltpu.sync_copy(adj_o, adj_out_hbm.at[pl.ds(row0, ROWS_W)])


def _sc_gather_agg(pu, v2e, prb, idxh, idxr, idxa, nodes):
    mesh = plsc.VectorSubcoreMesh(core_axis_name="c", subcore_axis_name="s")
    f32 = jnp.float32
    kern = functools.partial(
        pl.kernel,
        mesh=mesh,
        compiler_params=pltpu.CompilerParams(use_tc_tiling_on_sc=False,
                                             needs_layout_passes=False),
        out_type=[
            jax.ShapeDtypeStruct((B, D // 2), jnp.int32),
            jax.ShapeDtypeStruct((B, D // 2), jnp.int32),
            jax.ShapeDtypeStruct((B, D // 2), jnp.int32),
        ],
        scratch_types=[
            pltpu.VMEM((NCHUNK, CHUNK * LP), jnp.int32),
            pltpu.VMEM((NCHUNK, CHUNK * LP), jnp.int32),
            pltpu.VMEM((NCHUNK, CHUNK * LP), jnp.int32),
            pltpu.VMEM((ROWS_W,), jnp.int32),
            pltpu.VMEM((8 * (D // 2),), jnp.int32),
            pltpu.VMEM((CHUNK * LP, D // 2), jnp.int32),
            pltpu.VMEM((CHUNK * LP, D // 2), jnp.int32),
            pltpu.VMEM((CHUNK * LP, D // 2), jnp.int32),
            pltpu.VMEM((CHUNK * LP, D // 2), jnp.int32),
            pltpu.VMEM((ROWS_W, D // 2), jnp.int32),
            pltpu.VMEM((ROWS_W, D // 2), jnp.int32),
            pltpu.VMEM((ROWS_W, D // 2), jnp.int32),
            pltpu.SemaphoreType.DMA,
            pltpu.SemaphoreType.DMA,
            pltpu.SemaphoreType.DMA,
            pltpu.SemaphoreType.DMA,
            pltpu.SemaphoreType.DMA,
        ],
    )(_sc_body)
    return kern(pu, v2e, prb, idxh, idxr, idxa, nodes)


def _combine_body(self_ref, hist_ref, adj_ref, ws_ref, bs_ref,
                  w1a_ref, w1b_ref, b1_ref, out_ref):
    adj = adj_ref[...].astype(jnp.float32)
    hist = hist_ref[...].astype(jnp.float32)
    selff = self_ref[...].astype(jnp.float32)
    soc = jnp.maximum(
        jnp.dot(adj, ws_ref[...], preferred_element_type=jnp.float32)
        + bs_ref[...], 0.0)
    neigh = 0.5 * (hist + soc)
    out = (jnp.dot(selff, w1a_ref[...], preferred_element_type=jnp.float32)
           + jnp.dot(neigh, w1b_ref[...], preferred_element_type=jnp.float32)
           + b1_ref[...])
    out_ref[...] = jnp.maximum(out, 0.0)


def _combine(self_feats, hist_agg, adj_mean, Ws, bs_row, w1a, w1b, b1_row):
    return pl.pallas_call(
        _combine_body,
        out_shape=jax.ShapeDtypeStruct((B, D), jnp.float32),
    )(self_feats, hist_agg, adj_mean, Ws, bs_row, w1a, w1b, b1_row)


def kernel(nodes, nodes_target, hist_uv, hist_r, adj, u2e, v2e, r2e,
           Wh, bh, Ws, bs, W1, b1):
    del nodes_target

    wh1 = Wh[:D]
    wh2 = Wh[D:]
    w1a = W1[:D]
    w1b = W1[D:]
    r2e_pad = jnp.concatenate(
        [r2e, jnp.zeros((8 - r2e.shape[0], D), jnp.float32)], axis=0)
    bh_row = bh.reshape(1, D)
    bs_row = bs.reshape(1, D)
    b1_row = b1.reshape(1, D)

    pu, prb = _project_tables(u2e, wh1, r2e_pad, wh2, bh_row)

    def pad_lp(a):
        a = a.astype(jnp.int32)
        return jnp.pad(a, ((0, 0), (0, LP - L)))

    idxh = pad_lp(hist_uv).reshape(B // CHUNK, CHUNK * LP)
    idxa = pad_lp(adj).reshape(B // CHUNK, CHUNK * LP)
    idxr = (pad_lp(hist_r) * (D // 2)).reshape(B // CHUNK, CHUNK * LP)
    nodes_i = nodes.astype(jnp.int32)

    def to_i32(x_bf):
        return jax.lax.bitcast_convert_type(
            x_bf.reshape(x_bf.shape[0], D // 2, 2), jnp.int32)

    def from_i32(x_i32):
        return jax.lax.bitcast_convert_type(
            x_i32, jnp.bfloat16).reshape(x_i32.shape[0], D)

    hist_i, adj_i, self_i = _sc_gather_agg(
        to_i32(pu), to_i32(v2e.astype(jnp.bfloat16)), to_i32(prb).reshape(-1),
        idxh, idxr, idxa, nodes_i)

    return _combine(from_i32(self_i), from_i32(hist_i), from_i32(adj_i),
                    Ws, bs_row, w1a, w1b, b1_row)
